# baseline (speedup 1.0000x reference)
import functools

import numpy as np

import jax
import jax.numpy as jnp
from jax.experimental import pallas as pl
from jax.experimental.pallas import tpu as pltpu

_VMEM_LIMIT = 64 * 1024 * 1024


def _interp_matrix_np(out_size: int, in_size: int) -> np.ndarray:
    if in_size == 1:
        return np.ones((out_size, 1), np.float32)
    if out_size == 1:
        pos = np.zeros((1,), np.float64)
    else:
        pos = np.arange(out_size, dtype=np.float64) * (
            (in_size - 1) / (out_size - 1))
    lo = np.clip(np.floor(pos).astype(np.int64), 0, in_size - 2)
    frac = (pos - lo).astype(np.float32)
    m = np.zeros((out_size, in_size), np.float32)
    m[np.arange(out_size), lo] += 1.0 - frac
    m[np.arange(out_size), lo + 1] += frac
    return m


@functools.lru_cache(maxsize=None)
def _weights_np(in_h, in_w, out_h, out_w):
    wy = _interp_matrix_np(out_h, in_h)
    wxt = _interp_matrix_np(out_w, in_w).T
    return np.ascontiguousarray(wy), np.ascontiguousarray(wxt)


def _resize_kernel(wy_ref, wxt_ref, img_ref, out_ref, *, H, W, iH, iW, tm):
    wy = jnp.broadcast_to(wy_ref[...], (tm, iH, H))
    t1 = jax.lax.dot_general(wy, img_ref[...],
                             dimension_numbers=(((2,), (1,)), ((0,), (0,))),
                             preferred_element_type=jnp.float32)
    out = jnp.dot(t1.reshape(tm * iH, W), wxt_ref[...],
                  preferred_element_type=jnp.float32)
    out_ref[...] = out.reshape(tm, iH, iW)


def kernel(video_flat):
    M, H, W = video_flat.shape
    iH, iW = 24, 32
    tm = 18
    assert M % tm == 0

    wy_np, wxt_np = _weights_np(H, W, iH, iW)
    wy = jnp.asarray(wy_np)
    wxt = jnp.asarray(wxt_np)

    grid = (M // tm,)
    cost = pl.CostEstimate(
        flops=2 * M * H * W * iW + 2 * M * iH * H * iW,
        transcendentals=0,
        bytes_accessed=(M * H * W + M * iH * iW) * 4)
    out = pl.pallas_call(
        functools.partial(_resize_kernel, H=H, W=W, iH=iH, iW=iW, tm=tm),
        out_shape=jax.ShapeDtypeStruct((M, iH, iW), jnp.float32),
        grid=grid,
        in_specs=[
            pl.BlockSpec((iH, H), lambda g: (0, 0)),
            pl.BlockSpec((W, iW), lambda g: (0, 0)),
            pl.BlockSpec((tm, H, W), lambda g: (g, 0, 0)),
        ],
        out_specs=pl.BlockSpec((tm, iH, iW), lambda g: (g, 0, 0)),
        compiler_params=pltpu.CompilerParams(
            dimension_semantics=("parallel",),
            vmem_limit_bytes=_VMEM_LIMIT),
        cost_estimate=cost,
    )(wy, wxt, video_flat.astype(jnp.float32))
    return out


# speedup vs baseline: 1.0287x; 1.0287x over previous
import functools

import numpy as np

import jax
import jax.numpy as jnp
from jax.experimental import pallas as pl
from jax.experimental.pallas import tpu as pltpu

_VMEM_LIMIT = 64 * 1024 * 1024


def _interp_matrix_np(out_size: int, in_size: int) -> np.ndarray:
    if in_size == 1:
        return np.ones((out_size, 1), np.float32)
    if out_size == 1:
        pos = np.zeros((1,), np.float64)
    else:
        pos = np.arange(out_size, dtype=np.float64) * (
            (in_size - 1) / (out_size - 1))
    lo = np.clip(np.floor(pos).astype(np.int64), 0, in_size - 2)
    frac = (pos - lo).astype(np.float32)
    m = np.zeros((out_size, in_size), np.float32)
    m[np.arange(out_size), lo] += 1.0 - frac
    m[np.arange(out_size), lo + 1] += frac
    return m


@functools.lru_cache(maxsize=None)
def _weights_np(in_h, in_w, out_h, out_w):
    wy = _interp_matrix_np(out_h, in_h)
    wxt = _interp_matrix_np(out_w, in_w).T
    return np.ascontiguousarray(wy), np.ascontiguousarray(wxt)


def _resize_kernel(wy_ref, wxt_ref, img_ref, out_ref, *, H, W, iH, iW, tm):
    wy = jnp.broadcast_to(wy_ref[...], (tm, iH, H))
    t1 = jax.lax.dot_general(wy, img_ref[...],
                             dimension_numbers=(((2,), (1,)), ((0,), (0,))),
                             preferred_element_type=jnp.float32)
    out = jnp.dot(t1.reshape(tm * iH, W), wxt_ref[...],
                  preferred_element_type=jnp.float32)
    out_ref[...] = out.reshape(tm, iH, iW)


def kernel(video_flat):
    M, H, W = video_flat.shape
    iH, iW = 24, 32
    tm = 48
    assert M % tm == 0

    wy_np, wxt_np = _weights_np(H, W, iH, iW)
    wy = jnp.asarray(wy_np)
    wxt = jnp.asarray(wxt_np)

    grid = (M // tm,)
    cost = pl.CostEstimate(
        flops=2 * M * H * W * iW + 2 * M * iH * H * iW,
        transcendentals=0,
        bytes_accessed=(M * H * W + M * iH * iW) * 4)
    out = pl.pallas_call(
        functools.partial(_resize_kernel, H=H, W=W, iH=iH, iW=iW, tm=tm),
        out_shape=jax.ShapeDtypeStruct((M, iH, iW), jnp.float32),
        grid=grid,
        in_specs=[
            pl.BlockSpec((iH, H), lambda g: (0, 0)),
            pl.BlockSpec((W, iW), lambda g: (0, 0)),
            pl.BlockSpec((tm, H, W), lambda g: (g, 0, 0)),
        ],
        out_specs=pl.BlockSpec((tm, iH, iW), lambda g: (g, 0, 0)),
        compiler_params=pltpu.CompilerParams(
            dimension_semantics=("parallel",),
            vmem_limit_bytes=_VMEM_LIMIT),
        cost_estimate=cost,
    )(wy, wxt, video_flat.astype(jnp.float32))
    return out


# speedup vs baseline: 1.0680x; 1.0382x over previous
import functools

import numpy as np

import jax
import jax.numpy as jnp
from jax.experimental import pallas as pl
from jax.experimental.pallas import tpu as pltpu

_VMEM_LIMIT = 64 * 1024 * 1024


def _interp_matrix_np(out_size: int, in_size: int) -> np.ndarray:
    if in_size == 1:
        return np.ones((out_size, 1), np.float32)
    if out_size == 1:
        pos = np.zeros((1,), np.float64)
    else:
        pos = np.arange(out_size, dtype=np.float64) * (
            (in_size - 1) / (out_size - 1))
    lo = np.clip(np.floor(pos).astype(np.int64), 0, in_size - 2)
    frac = (pos - lo).astype(np.float32)
    m = np.zeros((out_size, in_size), np.float32)
    m[np.arange(out_size), lo] += 1.0 - frac
    m[np.arange(out_size), lo + 1] += frac
    return m


@functools.lru_cache(maxsize=None)
def _weights_np(in_h, in_w, out_h, out_w):
    wy = _interp_matrix_np(out_h, in_h)
    wxt = _interp_matrix_np(out_w, in_w).T
    return np.ascontiguousarray(wy), np.ascontiguousarray(wxt)


def _resize_kernel(wy_ref, wxt_ref, img_ref, out_ref, *, H, W, iH, iW, tm):
    wy = jnp.broadcast_to(wy_ref[...], (tm, iH, H))
    t1 = jax.lax.dot_general(wy, img_ref[...],
                             dimension_numbers=(((2,), (1,)), ((0,), (0,))),
                             preferred_element_type=jnp.float32)
    out = jnp.dot(t1.reshape(tm * iH, W), wxt_ref[...],
                  preferred_element_type=jnp.float32)
    out_ref[...] = out.reshape(tm, iH, iW)


def kernel(video_flat):
    M, H, W = video_flat.shape
    iH, iW = 24, 32
    tm = 24
    assert M % tm == 0

    wy_np, wxt_np = _weights_np(H, W, iH, iW)
    wy = jnp.asarray(wy_np)
    wxt = jnp.asarray(wxt_np)

    grid = (M // tm,)
    cost = pl.CostEstimate(
        flops=2 * M * H * W * iW + 2 * M * iH * H * iW,
        transcendentals=0,
        bytes_accessed=(M * H * W + M * iH * iW) * 4)
    out = pl.pallas_call(
        functools.partial(_resize_kernel, H=H, W=W, iH=iH, iW=iW, tm=tm),
        out_shape=jax.ShapeDtypeStruct((M, iH, iW), jnp.float32),
        grid=grid,
        in_specs=[
            pl.BlockSpec((iH, H), lambda g: (0, 0)),
            pl.BlockSpec((W, iW), lambda g: (0, 0)),
            pl.BlockSpec((tm, H, W), lambda g: (g, 0, 0)),
        ],
        out_specs=pl.BlockSpec((tm, iH, iW), lambda g: (g, 0, 0)),
        compiler_params=pltpu.CompilerParams(
            dimension_semantics=("parallel",),
            vmem_limit_bytes=_VMEM_LIMIT),
        cost_estimate=cost,
    )(wy, wxt, video_flat.astype(jnp.float32))
    return out
